# baseline (speedup 1.0000x reference)
import functools

import jax
import jax.numpy as jnp
from jax import lax
from jax.experimental import pallas as pl
from jax.experimental.pallas import tpu as pltpu


def _fused_kernel(x_ref, w1_ref, b1_ref, gamma_ref, beta_ref, w2_ref,
                  b2_ref, o_ref, h_scr, st_scr, *, nb0, tb0, tb1,
                  b_total, eps):
    s = pl.program_id(0)

    @pl.when(s < nb0)
    def _phase0():
        i = s
        xb = x_ref[...].astype(jnp.bfloat16)
        w1b = w1_ref[...].astype(jnp.bfloat16)
        h = lax.dot_general(w1b, xb, (((1,), (0,)), ((), ())),
                            preferred_element_type=jnp.float32)
        h = h + b1_ref[...].T
        h_scr[:, pl.ds(i * tb0, tb0)] = h.astype(jnp.bfloat16)
        ones = jnp.ones((tb0, 1), jnp.float32)
        s1 = lax.dot_general(h, ones, (((1,), (0,)), ((), ())),
                             preferred_element_type=jnp.float32)
        s2 = lax.dot_general(h * h, ones, (((1,), (0,)), ((), ())),
                             preferred_element_type=jnp.float32)
        st = jnp.concatenate([s1, s2], axis=1)

        @pl.when(i == 0)
        def _init():
            st_scr[...] = st

        @pl.when(i > 0)
        def _acc():
            st_scr[...] += st

    @pl.when(s >= nb0)
    def _phase1():
        j = s - nb0
        st = st_scr[...]
        inv_b = 1.0 / float(b_total)
        mean = st[:, 0:1] * inv_b
        var = jnp.maximum(st[:, 1:2] * inv_b - mean * mean, 0.0)
        sc = gamma_ref[...].T * lax.rsqrt(var + eps)
        tc = beta_ref[...].T - mean * sc
        hb = h_scr[:, pl.ds(j * tb1, tb1)]
        a = jnp.maximum(hb.astype(jnp.float32) * sc + tc, 0.0)
        w2b = w2_ref[...].astype(jnp.bfloat16)
        y = jnp.dot(w2b, a.astype(jnp.bfloat16),
                    preferred_element_type=jnp.float32)
        o_ref[...] = y + b2_ref[...].T


def kernel(x, w1, b1, gamma, beta, w2, b2, *, eps=1e-5):
    B = x.shape[0]
    In = x.size // B
    H = w1.shape[0]
    C = w2.shape[0]

    xt = x.reshape(B, In).T
    tb0 = min(2048, B)
    tb1 = min(1024, B)
    nb0 = B // tb0
    nb1 = B // tb1

    b1r = b1.reshape(1, H)
    gr = gamma.reshape(1, H)
    br = beta.reshape(1, H)
    b2r = b2.reshape(1, C)

    yt = pl.pallas_call(
        functools.partial(_fused_kernel, nb0=nb0, tb0=tb0, tb1=tb1,
                          b_total=B, eps=eps),
        out_shape=jax.ShapeDtypeStruct((C, B), x.dtype),
        grid=(nb0 + nb1,),
        in_specs=[pl.BlockSpec((In, tb0),
                               lambda s: (0, jnp.minimum(s, nb0 - 1))),
                  pl.BlockSpec((H, In), lambda s: (0, 0)),
                  pl.BlockSpec((1, H), lambda s: (0, 0)),
                  pl.BlockSpec((1, H), lambda s: (0, 0)),
                  pl.BlockSpec((1, H), lambda s: (0, 0)),
                  pl.BlockSpec((C, H), lambda s: (0, 0)),
                  pl.BlockSpec((1, C), lambda s: (0, 0))],
        out_specs=pl.BlockSpec((C, tb1),
                               lambda s: (0, jnp.maximum(s - nb0, 0))),
        scratch_shapes=[pltpu.VMEM((H, B), jnp.bfloat16),
                        pltpu.VMEM((H, 2), jnp.float32)],
        compiler_params=pltpu.CompilerParams(
            dimension_semantics=("arbitrary",)),
        cost_estimate=pl.CostEstimate(
            flops=2 * B * In * H + 2 * B * H * C,
            transcendentals=H,
            bytes_accessed=4 * B * In + 4 * In * H + 4 * H * C + 4 * B * C),
    )(xt, w1, b1r, gr, br, w2, b2r)
    return yt.T
